# baseline (speedup 1.0000x reference)
import functools

import jax
import jax.numpy as jnp
from jax import lax
from jax.experimental import pallas as pl
from jax.experimental.pallas import tpu as pltpu
from jax.experimental.pallas import tpu_sc as plsc

_N = 10000
_E = 320000
_D = 128
_L = 5
_NC = 2
_NS = 16
_NW = _NC * _NS
_B = 32
_PE = 11200
_CH = _PE // _B
_OWN = 312
_LAST = _N - 31 * _OWN
_AP = _N + 16

_mesh = plsc.VectorSubcoreMesh(core_axis_name="c", subcore_axis_name="s")


@functools.partial(
    pl.kernel,
    out_type=jax.ShapeDtypeStruct((_N, 8, 16), jnp.float32),
    mesh=_mesh,
    scratch_types=[
        pltpu.VMEM((_CH // 2, _B), jnp.int32),
        pltpu.VMEM((_CH // 2, _B), jnp.int32),
        pltpu.VMEM((_CH // 2, _B), jnp.int32),
        pltpu.VMEM((_B, 8, 16), jnp.float32),
        pltpu.VMEM((_B, 8, 16), jnp.float32),
        pltpu.VMEM_SHARED((_AP, 8, 16), jnp.float32),
        pltpu.SemaphoreType.DMA,
        pltpu.SemaphoreType.DMA,
    ],
    compiler_params=pltpu.CompilerParams(use_tc_tiling_on_sc=False),
)
def _spmm_kernel(h_hbm, ee_hbm, row_hbm, col_hbm, pair_hbm, zero_hbm,
                 out_hbm, row_v, col_v, pair_v, hv, ev, acc, sem, sem2):
    c = lax.axis_index("c")
    s = lax.axis_index("s")
    wid = s * _NC + c
    base = wid * _OWN

    @pl.when(wid < _NW - 1)
    def _():
        pltpu.sync_copy(zero_hbm.at[pl.ds(0, _OWN)],
                        acc.at[pl.ds(base, _OWN)])

    @pl.when(wid == _NW - 1)
    def _():
        pltpu.sync_copy(zero_hbm, acc.at[pl.ds(base, _LAST + 16)])
    plsc.subcore_barrier()

    def chunk(j, carry):
        cp_h = pltpu.async_copy(h_hbm.at[row_v.at[j]], hv, sem)
        cp_e = pltpu.async_copy(ee_hbm.at[pair_v.at[j]], ev, sem2)
        cp_h.wait()
        cp_e.wait()

        for b in range(_B):
            for k in range(8):
                hv[b, k] = hv[b, k] + ev[b, k]
        pltpu.sync_copy(hv, acc.at[col_v.at[j]], add=True)
        return carry

    for half in range(2):
        hs = half * (_CH // 2)
        pltpu.sync_copy(row_hbm.at[wid, pl.ds(hs, _CH // 2)], row_v)
        pltpu.sync_copy(col_hbm.at[wid, pl.ds(hs, _CH // 2)], col_v)
        pltpu.sync_copy(pair_hbm.at[wid, pl.ds(hs, _CH // 2)], pair_v)
        lax.fori_loop(0, _CH // 2, chunk, 0)
    plsc.subcore_barrier()

    @pl.when(wid < _NW - 1)
    def _():
        pltpu.sync_copy(acc.at[pl.ds(base, _OWN)],
                        out_hbm.at[pl.ds(base, _OWN)])

    @pl.when(wid == _NW - 1)
    def _():
        pltpu.sync_copy(acc.at[pl.ds(base, _LAST)],
                        out_hbm.at[pl.ds(base, _LAST)])


_NB = 10
_BR = _N // _NB


def _prep_body(x_ref, xe1_ref, xe2_ref, h0_ref):
    xb = x_ref[...]
    oh1 = (xb[:, 0:1] == lax.broadcasted_iota(jnp.int32, (_BR, 120), 1))
    oh2 = (xb[:, 1:2] == lax.broadcasted_iota(jnp.int32, (_BR, 4), 1))
    h0 = jnp.dot(oh1.astype(jnp.float32), xe1_ref[...],
                 preferred_element_type=jnp.float32,
                 precision=lax.Precision.HIGHEST)
    h0_ref[...] = h0 + jnp.dot(oh2.astype(jnp.float32), xe2_ref[...],
                               preferred_element_type=jnp.float32,
                               precision=lax.Precision.HIGHEST)


_prep_call = pl.pallas_call(
    _prep_body,
    grid=(_NB,),
    in_specs=[
        pl.BlockSpec((_BR, 2), lambda i: (i, 0)),
        pl.BlockSpec((120, _D), lambda i: (0, 0)),
        pl.BlockSpec((4, _D), lambda i: (0, 0)),
    ],
    out_specs=pl.BlockSpec((_BR, _D), lambda i: (i, 0)),
    out_shape=jax.ShapeDtypeStruct((_N, _D), jnp.float32),
)


def _bn(h, gamma, beta, eps=1e-5):
    mu = jnp.mean(h, axis=0, keepdims=True)
    var = jnp.var(h, axis=0, keepdims=True)
    return gamma * (h - mu) / jnp.sqrt(var + eps) + beta


def kernel(x, edge_index, edge_attr, xemb1, xemb2, eemb1, eemb2,
           W1, b1, g1, beta1, W2, b2, go, bo):
    row = edge_index[0]
    col = edge_index[1]
    pair = edge_attr[:, 0] * 3 + edge_attr[:, 1]

    owner = jnp.minimum(col // _OWN, _NW - 1)
    sidx = jnp.argsort(owner, stable=True)
    ow_s = owner[sidx]
    first = jnp.searchsorted(ow_s, ow_s, side="left")
    off = jnp.arange(_E, dtype=first.dtype) - first
    ok = off < _PE
    dest = jnp.where(ok, ow_s * _PE + off, _NW * _PE)
    row_p = jnp.zeros((_NW * _PE + 1,), jnp.int32).at[dest].set(row[sidx])
    col_p = jnp.full((_NW * _PE + 1,), _N, jnp.int32).at[dest].set(col[sidx])
    pair_p = jnp.zeros((_NW * _PE + 1,), jnp.int32).at[dest].set(pair[sidx])
    row_p = row_p[:-1].reshape(_NW, _CH, _B)
    col_p = col_p[:-1].reshape(_NW, _CH, _B)
    pair_p = pair_p[:-1].reshape(_NW, _CH, _B)

    ee15 = (eemb1[:, :, None, :] + eemb2[:, None, :, :]).reshape(_L, 15, 8, 16)
    zero = jnp.zeros((_LAST + 16, 8, 16), jnp.float32)

    h = _prep_call(x, xemb1, xemb2)

    for l in range(_L):
        p = _spmm_kernel(h.reshape(_N, 8, 16), ee15[l], row_p, col_p,
                         pair_p, zero)
        agg = p.reshape(_N, _D)
        z = agg @ W1[l] + b1[l]
        z = _bn(z, g1[l], beta1[l])
        z = jax.nn.relu(z)
        h = z @ W2[l] + b2[l]
        h = _bn(h, go[l], bo[l])
        if l < _L - 1:
            h = jax.nn.relu(h)
    return h
